# baseline (speedup 1.0000x reference)
import functools

import jax
import jax.numpy as jnp
from jax import lax
from jax.experimental import pallas as pl
from jax.experimental.pallas import tpu as pltpu
from jax.experimental.pallas import tpu_sc as plsc

N_WORDS = 100000
EMBED_DIM = 300
BATCH = 16384

_info = plsc.get_sparse_core_info()
_NC = _info.num_cores
_NS = _info.num_subcores
_NW = _NC * _NS
_BPW = BATCH // _NW
_CHUNK = 128
_NCHUNK = _BPW // _CHUNK
_LANES = 16

_mesh = plsc.VectorSubcoreMesh(core_axis_name="c", subcore_axis_name="s")


@functools.partial(
    pl.kernel,
    mesh=_mesh,
    compiler_params=pltpu.CompilerParams(use_tc_tiling_on_sc=False),
    out_type=[jax.ShapeDtypeStruct((BATCH, EMBED_DIM), jnp.float32)] * 3,
    scratch_types=[
        pltpu.VMEM((_NCHUNK, _CHUNK), jnp.int32),
        pltpu.VMEM((_CHUNK, EMBED_DIM), jnp.float32),
        pltpu.SemaphoreType.DMA,
    ],
)
def _embed3(tw_hbm, syn_hbm, ant_hbm, table_hbm,
            out_tw, out_syn, out_ant,
            idx_v, rows, sem):
    wid = lax.axis_index("s") * _NC + lax.axis_index("c")
    base = wid * _BPW
    for idx_hbm, out_hbm in (
        (tw_hbm, out_tw),
        (syn_hbm, out_syn),
        (ant_hbm, out_ant),
    ):
        pltpu.sync_copy(idx_hbm.at[wid], idx_v)
        for c in range(_NCHUNK):
            @pl.loop(0, _CHUNK // _LANES)
            def _fire(g, c=c):
                vec = idx_v[c, pl.ds(g * _LANES, _LANES)]
                for j in range(_LANES):
                    pltpu.make_async_copy(
                        table_hbm.at[pl.ds(vec[j], 1)],
                        rows.at[pl.ds(g * _LANES + j, 1)],
                        sem,
                    ).start()
            pltpu.make_async_copy(
                table_hbm.at[pl.ds(0, _CHUNK)], rows, sem
            ).wait()
            pltpu.sync_copy(rows, out_hbm.at[pl.ds(base + c * _CHUNK, _CHUNK)])


def kernel(target_word, synonym, antonym, embedding_weight):
    tw = target_word.astype(jnp.int32).reshape(_NW, _NCHUNK, _CHUNK)
    syn = synonym.astype(jnp.int32).reshape(_NW, _NCHUNK, _CHUNK)
    ant = antonym.astype(jnp.int32).reshape(_NW, _NCHUNK, _CHUNK)
    o = _embed3(tw, syn, ant, embedding_weight)
    return (o[0], o[1], o[2])


# speedup vs baseline: 1.0061x; 1.0061x over previous
import functools

import jax
import jax.numpy as jnp
from jax import lax
from jax.experimental import pallas as pl
from jax.experimental.pallas import tpu as pltpu
from jax.experimental.pallas import tpu_sc as plsc

N_WORDS = 100000
EMBED_DIM = 300
PAD_DIM = 304
BATCH = 16384

_info = plsc.get_sparse_core_info()
_NC = _info.num_cores
_NS = _info.num_subcores
_NW = _NC * _NS
_BPW = BATCH // _NW
_CHUNK = 128
_NCHUNK = _BPW // _CHUNK
_NT = 3 * _NCHUNK

_mesh = plsc.VectorSubcoreMesh(core_axis_name="c", subcore_axis_name="s")


@functools.partial(
    pl.kernel,
    mesh=_mesh,
    compiler_params=pltpu.CompilerParams(use_tc_tiling_on_sc=False),
    out_type=[jax.ShapeDtypeStruct((BATCH, PAD_DIM), jnp.float32)] * 3,
    scratch_types=[
        pltpu.VMEM((_NT, _CHUNK), jnp.int32),
        pltpu.VMEM((_CHUNK, PAD_DIM), jnp.float32),
        pltpu.VMEM((_CHUNK, PAD_DIM), jnp.float32),
        pltpu.SemaphoreType.DMA,
        pltpu.SemaphoreType.DMA,
    ],
)
def _embed3(idx_hbm, table_hbm, out_tw, out_syn, out_ant,
            idx_v, rows0, rows1, sem0, sem1):
    wid = lax.axis_index("s") * _NC + lax.axis_index("c")
    base = wid * _BPW
    pltpu.sync_copy(idx_hbm.at[wid], idx_v)
    outs = (out_tw, out_syn, out_ant)
    rows = (rows0, rows1)
    sems = (sem0, sem1)

    def fire(t):
        cp = pltpu.make_async_copy(
            table_hbm.at[idx_v.at[t]], rows[t % 2], sems[t % 2]
        )
        cp.start()
        return cp

    cp = fire(0)
    for t in range(_NT):
        cp.wait()
        if t + 1 < _NT:
            nxt = fire(t + 1)
        out_hbm = outs[t // _NCHUNK]
        off = base + (t % _NCHUNK) * _CHUNK
        pltpu.sync_copy(rows[t % 2], out_hbm.at[pl.ds(off, _CHUNK)])
        if t + 1 < _NT:
            cp = nxt


def kernel(target_word, synonym, antonym, embedding_weight):
    idx = jnp.stack(
        [target_word.astype(jnp.int32),
         synonym.astype(jnp.int32),
         antonym.astype(jnp.int32)]
    )
    idx = (
        idx.reshape(3, _NW, _NCHUNK, _CHUNK)
        .transpose(1, 0, 2, 3)
        .reshape(_NW, _NT, _CHUNK)
    )
    tab = jnp.pad(embedding_weight, ((0, 0), (0, PAD_DIM - EMBED_DIM)))
    o = _embed3(idx, tab)
    return (o[0][:, :EMBED_DIM], o[1][:, :EMBED_DIM], o[2][:, :EMBED_DIM])


# speedup vs baseline: 1.4015x; 1.3929x over previous
import functools

import jax
import jax.numpy as jnp
from jax import lax
from jax.experimental import pallas as pl
from jax.experimental.pallas import tpu as pltpu
from jax.experimental.pallas import tpu_sc as plsc

N_WORDS = 100000
EMBED_DIM = 300
PAD_DIM = 304
BATCH = 16384

_info = plsc.get_sparse_core_info()
_NC = _info.num_cores
_NS = _info.num_subcores
_NW = _NC * _NS
_BPW = BATCH // _NW
_CHUNK = 128
_NCHUNK = _BPW // _CHUNK
_NT = 3 * _NCHUNK

_mesh = plsc.VectorSubcoreMesh(core_axis_name="c", subcore_axis_name="s")

_PAD_ROWS = 1000
_UNPAD_ROWS = 512


def _pad_body(x_ref, o_ref):
    o_ref[:, :EMBED_DIM] = x_ref[...]
    o_ref[:, EMBED_DIM:] = jnp.zeros(
        (_PAD_ROWS, PAD_DIM - EMBED_DIM), jnp.float32
    )


_pad_table = pl.pallas_call(
    _pad_body,
    grid=(N_WORDS // _PAD_ROWS,),
    in_specs=[pl.BlockSpec((_PAD_ROWS, EMBED_DIM), lambda i: (i, 0))],
    out_specs=pl.BlockSpec((_PAD_ROWS, PAD_DIM), lambda i: (i, 0)),
    out_shape=jax.ShapeDtypeStruct((N_WORDS, PAD_DIM), jnp.float32),
)


def _unpad_body(a_ref, b_ref, c_ref, oa_ref, ob_ref, oc_ref):
    oa_ref[...] = a_ref[:, :EMBED_DIM]
    ob_ref[...] = b_ref[:, :EMBED_DIM]
    oc_ref[...] = c_ref[:, :EMBED_DIM]


_unpad3 = pl.pallas_call(
    _unpad_body,
    grid=(BATCH // _UNPAD_ROWS,),
    in_specs=[pl.BlockSpec((_UNPAD_ROWS, PAD_DIM), lambda i: (i, 0))] * 3,
    out_specs=[pl.BlockSpec((_UNPAD_ROWS, EMBED_DIM), lambda i: (i, 0))] * 3,
    out_shape=[jax.ShapeDtypeStruct((BATCH, EMBED_DIM), jnp.float32)] * 3,
)


@functools.partial(
    pl.kernel,
    mesh=_mesh,
    compiler_params=pltpu.CompilerParams(use_tc_tiling_on_sc=False),
    out_type=[jax.ShapeDtypeStruct((BATCH, PAD_DIM), jnp.float32)] * 3,
    scratch_types=[
        pltpu.VMEM((_NT, _CHUNK), jnp.int32),
        pltpu.VMEM((_CHUNK, PAD_DIM), jnp.float32),
        pltpu.VMEM((_CHUNK, PAD_DIM), jnp.float32),
        pltpu.SemaphoreType.DMA,
        pltpu.SemaphoreType.DMA,
    ],
)
def _embed3(idx_hbm, table_hbm, out_tw, out_syn, out_ant,
            idx_v, rows0, rows1, sem0, sem1):
    wid = lax.axis_index("s") * _NC + lax.axis_index("c")
    base = wid * _BPW
    pltpu.sync_copy(idx_hbm.at[wid], idx_v)
    outs = (out_tw, out_syn, out_ant)
    rows = (rows0, rows1)
    sems = (sem0, sem1)

    def fire(t):
        cp = pltpu.make_async_copy(
            table_hbm.at[idx_v.at[t]], rows[t % 2], sems[t % 2]
        )
        cp.start()
        return cp

    cp = fire(0)
    for t in range(_NT):
        cp.wait()
        if t + 1 < _NT:
            nxt = fire(t + 1)
        out_hbm = outs[t // _NCHUNK]
        off = base + (t % _NCHUNK) * _CHUNK
        pltpu.sync_copy(rows[t % 2], out_hbm.at[pl.ds(off, _CHUNK)])
        if t + 1 < _NT:
            cp = nxt


def kernel(target_word, synonym, antonym, embedding_weight):
    idx = jnp.stack(
        [target_word.astype(jnp.int32),
         synonym.astype(jnp.int32),
         antonym.astype(jnp.int32)]
    )
    idx = (
        idx.reshape(3, _NW, _NCHUNK, _CHUNK)
        .transpose(1, 0, 2, 3)
        .reshape(_NW, _NT, _CHUNK)
    )
    tab = _pad_table(embedding_weight)
    o = _embed3(idx, tab)
    return tuple(_unpad3(o[0], o[1], o[2]))


# speedup vs baseline: 1.5929x; 1.1366x over previous
import functools

import jax
import jax.numpy as jnp
from jax import lax
from jax.experimental import pallas as pl
from jax.experimental.pallas import tpu as pltpu
from jax.experimental.pallas import tpu_sc as plsc

N_WORDS = 100000
EMBED_DIM = 300
PAD_DIM = 304
BATCH = 16384

_info = plsc.get_sparse_core_info()
_NC = _info.num_cores
_NS = _info.num_subcores
_NW = _NC * _NS
_BPW = BATCH // _NW
_CHUNK = 128
_NCHUNK = _BPW // _CHUNK
_NT = 3 * _NCHUNK

_mesh = plsc.VectorSubcoreMesh(core_axis_name="c", subcore_axis_name="s")

_PAD_ROWS = 1000


def _pad_body(x_ref, o_ref):
    o_ref[:, :EMBED_DIM] = x_ref[...]
    o_ref[:, EMBED_DIM:] = jnp.zeros(
        (_PAD_ROWS, PAD_DIM - EMBED_DIM), jnp.float32
    )


_pad_table = pl.pallas_call(
    _pad_body,
    grid=(N_WORDS // _PAD_ROWS,),
    in_specs=[pl.BlockSpec((_PAD_ROWS, EMBED_DIM), lambda i: (i, 0))],
    out_specs=pl.BlockSpec((_PAD_ROWS, PAD_DIM), lambda i: (i, 0)),
    out_shape=jax.ShapeDtypeStruct((N_WORDS, PAD_DIM), jnp.float32),
)


@functools.partial(
    pl.kernel,
    mesh=_mesh,
    compiler_params=pltpu.CompilerParams(use_tc_tiling_on_sc=False),
    out_type=[jax.ShapeDtypeStruct((BATCH, EMBED_DIM), jnp.float32)] * 3,
    scratch_types=[
        pltpu.VMEM((_NT, _CHUNK), jnp.int32),
        pltpu.VMEM((_CHUNK, PAD_DIM), jnp.float32),
        pltpu.VMEM((_CHUNK, PAD_DIM), jnp.float32),
        pltpu.VMEM((_CHUNK, EMBED_DIM), jnp.float32),
        pltpu.SemaphoreType.DMA,
        pltpu.SemaphoreType.DMA,
        pltpu.SemaphoreType.DMA,
        pltpu.SemaphoreType.DMA,
    ],
)
def _embed3(idx_hbm, table_hbm, out_tw, out_syn, out_ant,
            idx_v, rows0, rows1, buf, sem0, sem1, semw1, semw2):
    wid = lax.axis_index("s") * _NC + lax.axis_index("c")
    base = wid * _BPW
    pltpu.sync_copy(idx_hbm.at[wid], idx_v)
    outs = (out_tw, out_syn, out_ant)
    rows = (rows0, rows1)
    sems = (sem0, sem1)

    def fire(t):
        cp = pltpu.make_async_copy(
            table_hbm.at[idx_v.at[t]], rows[t % 2], sems[t % 2]
        )
        cp.start()
        return cp

    cp = fire(0)
    for t in range(_NT):
        cp.wait()
        if t + 1 < _NT:
            nxt = fire(t + 1)
        src = rows[t % 2]
        out_hbm = outs[t // _NCHUNK]
        off = base + (t % _NCHUNK) * _CHUNK

        @pl.loop(0, _CHUNK)
        def _repack(k):
            for j in range(18):
                buf[k, pl.ds(j * 16, 16)] = src[k, pl.ds(j * 16, 16)]
            buf[k, pl.ds(284, 16)] = src[k, pl.ds(284, 16)]

        pltpu.sync_copy(buf, out_hbm.at[pl.ds(off, _CHUNK)])
        if t + 1 < _NT:
            cp = nxt


def kernel(target_word, synonym, antonym, embedding_weight):
    idx = jnp.stack(
        [target_word.astype(jnp.int32),
         synonym.astype(jnp.int32),
         antonym.astype(jnp.int32)]
    )
    idx = (
        idx.reshape(3, _NW, _NCHUNK, _CHUNK)
        .transpose(1, 0, 2, 3)
        .reshape(_NW, _NT, _CHUNK)
    )
    tab = _pad_table(embedding_weight)
    o = _embed3(idx, tab)
    return (o[0], o[1], o[2])
